# Initial kernel scaffold; baseline (speedup 1.0000x reference)
import jax
import jax.numpy as jnp
from jax.experimental import pallas as pl


def kernel(params, user, item, user_self_cate, user_onehop_id, user_onehop_cate, item_self_cate, item_self_director, item_self_writer, item_self_star, item_self_country, item_onehop_id, item_onehop_cate, item_onehop_director, item_onehop_writer, item_onehop_star, item_onehop_country):
    raise NotImplementedError("write your pallas kernel here")


# speedup vs baseline: 2.2183x; 2.2183x over previous
import functools

import jax
import jax.numpy as jnp
from jax import lax
from jax.experimental import pallas as pl
from jax.experimental.pallas import tpu as pltpu
from jax.experimental.pallas import tpu_sc as plsc

_NC, _NS = 2, 16
_NW = _NC * _NS
_CH = 128

_LAYERS = (
    'dense_item_self_biinter', 'dense_item_self_siinter',
    'dense_item_onehop_biinter', 'dense_item_onehop_siinter',
    'dense_user_self_biinter', 'dense_user_self_siinter',
    'dense_user_onehop_biinter', 'dense_user_onehop_siinter',
    'dense_item_cate_self', 'dense_item_cate_hop1',
    'dense_user_cate_self', 'dense_user_cate_hop1',
    'dense_item_addgate', 'dense_item_erasegate',
    'dense_user_addgate', 'dense_user_erasegate',
    'FC_pre',
)


def _sc_gather(dwsc_tab, emb_tab, user_tab, ubias, ibias,
               dwsc_idx, emb_idx, uidx, iidx):
    r_dwsc = dwsc_idx.shape[0]
    r_emb = emb_idx.shape[0]
    b = uidx.shape[0]
    e = emb_tab.shape[1]
    n_dwsc = r_dwsc // (_NW * _CH)
    n_emb = r_emb // (_NW * _CH)
    n_u = b // (_NW * _CH)

    mesh = plsc.VectorSubcoreMesh(core_axis_name="c", subcore_axis_name="s",
                                  num_cores=_NC, num_subcores=_NS)

    @functools.partial(
        pl.kernel,
        out_type=[
            jax.ShapeDtypeStruct((r_dwsc, e), jnp.float32),
            jax.ShapeDtypeStruct((r_emb, e), jnp.float32),
            jax.ShapeDtypeStruct((b, e), jnp.float32),
            jax.ShapeDtypeStruct((b,), jnp.float32),
            jax.ShapeDtypeStruct((b,), jnp.float32),
        ],
        mesh=mesh,
        compiler_params=pltpu.CompilerParams(use_tc_tiling_on_sc=False),
        scratch_types=[
            pltpu.VMEM((_CH,), jnp.int32),
            pltpu.VMEM((_CH, e), jnp.float32),
            pltpu.VMEM((_CH,), jnp.float32),
            pltpu.SemaphoreType.DMA,
        ],
    )
    def run(dwsc_ref, emb_ref, user_ref, ub_ref, ib_ref,
            dwsc_i_ref, emb_i_ref, u_i_ref, i_i_ref,
            dwsc_o, emb_o, urow_o, ubg_o, ibg_o,
            idx_v, rows_v, val_v, sem):
        wid = lax.axis_index("s") * _NC + lax.axis_index("c")

        def row_gather(tab, idx_hbm, out_hbm, nchunk):
            base0 = wid * (nchunk * _CH)

            def body(i, carry):
                off = base0 + i * _CH
                pltpu.sync_copy(idx_hbm.at[pl.ds(off, _CH)], idx_v)
                pltpu.async_copy(tab.at[idx_v], rows_v, sem).wait()
                pltpu.sync_copy(rows_v, out_hbm.at[pl.ds(off, _CH)])
                return carry

            lax.fori_loop(0, nchunk, body, 0)

        row_gather(dwsc_ref, dwsc_i_ref, dwsc_o, n_dwsc)
        row_gather(emb_ref, emb_i_ref, emb_o, n_emb)
        row_gather(user_ref, u_i_ref, urow_o, n_u)

        def elem_gather(tab, idx_hbm, out_hbm):
            off = wid * _CH
            pltpu.sync_copy(idx_hbm.at[pl.ds(off, _CH)], idx_v)
            pltpu.async_copy(tab.at[idx_v], val_v, sem).wait()
            pltpu.sync_copy(val_v, out_hbm.at[pl.ds(off, _CH)])

        elem_gather(ub_ref, u_i_ref, ubg_o)
        elem_gather(ib_ref, i_i_ref, ibg_o)

    return run(dwsc_tab, emb_tab, user_tab, ubias, ibias,
               dwsc_idx, emb_idx, uidx, iidx)


def _tc_forward(u_rows, ie, inb, unb, dws_self, dws_hop, ub2, ib2,
                isc, ioc, usc, uoc, genre, gender, age, occ, miu2,
                wb, B, NB, E):
    BB = 128
    G = B // BB
    P = BB * NB

    def body(u_ref, ie_ref, inb_ref, unb_ref, ds_ref, dh_ref, ub_ref, ib_ref,
             isc_ref, ioc_ref, usc_ref, uoc_ref, g_ref, gen_ref, age_ref,
             occ_ref, miu_ref, *rest):
        out_ref = rest[-1]
        wrefs = rest[:-1]
        wd = {}
        for k, name in enumerate(_LAYERS):
            wd[name] = (wrefs[2 * k][...], wrefs[2 * k + 1][...])

        def dot(x, w):
            return jax.lax.dot_general(
                x, w, (((1,), (0,)), ((), ())),
                precision=jax.lax.Precision.HIGHEST,
                preferred_element_type=jnp.float32)

        def lin(x, name):
            w, bb = wd[name]
            return dot(x, w) + bb

        def lin2(x1, x2, name):
            w, bb = wd[name]
            return dot(x1, w[:E, :]) + dot(x2, w[E:, :]) + bb

        def leaky(x):
            return jnp.where(x >= 0, x, 0.01 * x)

        def sig(x):
            return 1.0 / (1.0 + jnp.exp(-x))

        def interact(S, Q, bi, si):
            deep = 0.5 * (S * S - Q)
            return leaky(lin(deep, bi)) + leaky(lin(S, si))

        def sum8(v, npair):
            r = v.reshape(npair, 8, E)
            return r.sum(axis=1), (r * r).sum(axis=1)

        def onehot_counts(idx, V):
            n, k = idx.shape
            io = lax.broadcasted_iota(jnp.int32, (n, V), 1)
            c = jnp.zeros((n, V), jnp.float32)
            for s in range(k):
                c = c + (idx[:, s:s + 1] == io).astype(jnp.float32)
            return c

        g = g_ref[...]
        g2 = g * g
        NG = g.shape[0]

        Sd, Qd = sum8(ds_ref[...], BB)
        cg = onehot_counts(isc_ref[...], NG)
        att1 = interact(Sd + dot(cg, g), Qd + dot(cg, g2),
                        'dense_item_self_biinter', 'dense_item_self_siinter')
        item_cat = jnp.maximum(lin2(ie_ref[...], att1, 'dense_item_cate_self'), 0.0)

        Sh, Qh = sum8(dh_ref[...], P)
        chg = onehot_counts(ioc_ref[...], NG)
        att2 = interact(Sh + dot(chg, g), Qh + dot(chg, g2),
                        'dense_item_onehop_biinter', 'dense_item_onehop_siinter')
        item_nb = jnp.maximum(lin2(inb_ref[...], att2, 'dense_item_cate_hop1'), 0.0)
        item_nb_agg = item_nb.reshape(BB, NB, E).sum(axis=1) * (1.0 / NB)

        add_g = sig(lin2(item_cat, item_nb_agg, 'dense_item_addgate'))
        erase_g = sig(lin2(item_cat, item_nb_agg, 'dense_item_erasegate'))
        item_final = item_cat * (1.0 - erase_g) + item_nb_agg * add_g

        usc_v = usc_ref[...]

        def one_tab(idxcol, tab):
            V = tab.shape[0]
            io = lax.broadcasted_iota(jnp.int32, (BB, V), 1)
            oh = (idxcol == io).astype(jnp.float32)
            return dot(oh, tab), dot(oh, tab * tab)

        Sg, Qg = one_tab(usc_v[:, 0:1], gen_ref[...])
        Sa, Qa = one_tab(usc_v[:, 1:2], age_ref[...])
        So, Qo = one_tab(usc_v[:, 2:3], occ_ref[...])
        att3 = interact(Sg + Sa + So, Qg + Qa + Qo,
                        'dense_user_self_biinter', 'dense_user_self_siinter')
        user_cat = jnp.maximum(lin2(u_ref[...], att3, 'dense_user_cate_self'), 0.0)

        cu = onehot_counts(uoc_ref[...], NG)
        att4 = interact(dot(cu, g), dot(cu, g2),
                        'dense_user_onehop_biinter', 'dense_user_onehop_siinter')
        user_nb = jnp.maximum(lin2(unb_ref[...], att4, 'dense_user_cate_hop1'), 0.0)
        user_nb_agg = user_nb.reshape(BB, NB, E).sum(axis=1) * (1.0 / NB)

        uadd = sig(lin2(user_cat, user_nb_agg, 'dense_user_addgate'))
        uerase = sig(lin2(user_cat, user_nb_agg, 'dense_user_erasegate'))
        user_final = user_cat * (1.0 - uerase) + user_nb_agg * uadd

        wfc, bfc = wd['FC_pre']
        pred = ((user_final * wfc[:, :E]).sum(axis=1)
                + (item_final * wfc[:, E:]).sum(axis=1))
        pred = pred + bfc[0, 0] + miu_ref[0, 0] + ub_ref[...] + ib_ref[...]
        out_ref[...] = pred

    def blk(shape):
        return pl.BlockSpec(shape, lambda i: (i, 0))

    def full(a):
        return pl.BlockSpec(a.shape, lambda i: (0, 0))

    in_specs = [
        blk((BB, E)), blk((BB, E)), blk((P, E)), blk((P, E)),
        blk((BB * 8, E)), blk((P * 8, E)),
        pl.BlockSpec((BB,), lambda i: (i,)), pl.BlockSpec((BB,), lambda i: (i,)),
        blk((BB, 3)), blk((P, 3)), blk((BB, 3)), blk((P, 3)),
        full(genre), full(gender), full(age), full(occ), full(miu2),
    ]
    ops = [u_rows, ie, inb, unb, dws_self, dws_hop, ub2, ib2,
           isc, ioc, usc, uoc, genre, gender, age, occ, miu2]
    for w, bb_ in wb:
        in_specs.append(full(w))
        in_specs.append(full(bb_))
        ops.append(w)
        ops.append(bb_)

    out = pl.pallas_call(
        body,
        grid=(G,),
        in_specs=in_specs,
        out_specs=pl.BlockSpec((BB,), lambda i: (i,)),
        out_shape=jax.ShapeDtypeStruct((B,), jnp.float32),
    )(*ops)
    return out


def kernel(params, user, item, user_self_cate, user_onehop_id, user_onehop_cate,
           item_self_cate, item_self_director, item_self_writer, item_self_star,
           item_self_country, item_onehop_id, item_onehop_cate,
           item_onehop_director, item_onehop_writer, item_onehop_star,
           item_onehop_country):
    p = params
    B = user.shape[0]
    NB = user_onehop_id.shape[1]
    E = p['user_embed'].shape[1]

    d_tab = p['director_embed']
    w_tab = p['writer_embed']
    s_tab = p['star_embed']
    c_tab = p['country_embed']
    off_w = d_tab.shape[0]
    off_s = off_w + w_tab.shape[0]
    off_c = off_s + s_tab.shape[0]
    dwsc_tab = jnp.concatenate([d_tab, w_tab, s_tab, c_tab], axis=0)

    self8 = jnp.concatenate(
        [item_self_director, item_self_writer + off_w,
         item_self_star + off_s, item_self_country + off_c], axis=1)
    hop8 = jnp.concatenate(
        [item_onehop_director, item_onehop_writer + off_w,
         item_onehop_star + off_s, item_onehop_country + off_c], axis=2)
    dwsc_idx = jnp.concatenate([self8.reshape(-1), hop8.reshape(-1)])
    emb_idx = jnp.concatenate(
        [item, item_onehop_id.reshape(-1), user_onehop_id.reshape(-1)])

    dwsc_rows, emb_rows, u_rows, ub_g, ib_g = _sc_gather(
        dwsc_tab, p['item_embed'], p['user_embed'],
        p['user_bias'].reshape(-1), p['item_bias'].reshape(-1),
        dwsc_idx, emb_idx, user, item)

    ie = emb_rows[:B]
    inb = emb_rows[B:B + B * NB]
    unb = emb_rows[B + B * NB:]
    ds = dwsc_rows[:B * 8]
    dh = dwsc_rows[B * 8:]

    wb = []
    for name in _LAYERS[:-1]:
        wb.append((p[name + '_w'], p[name + '_b'].reshape(1, -1)))
    wb.append((p['FC_pre_w'].reshape(1, 2 * E), p['FC_pre_b'].reshape(1, 1)))

    out2 = _tc_forward(
        u_rows, ie, inb, unb, ds, dh, ub_g, ib_g,
        item_self_cate, item_onehop_cate.reshape(B * NB, 3),
        user_self_cate, user_onehop_cate.reshape(B * NB, 3),
        p['genre_embed'], p['gender_embed'], p['age_embed'],
        p['occupation_embed'], p['miu'].reshape(1, 1),
        wb, B, NB, E)
    return out2


# speedup vs baseline: 2.4265x; 1.0939x over previous
import functools

import jax
import jax.numpy as jnp
from jax import lax
from jax.experimental import pallas as pl
from jax.experimental.pallas import tpu as pltpu
from jax.experimental.pallas import tpu_sc as plsc

_NC, _NS = 2, 16
_NW = _NC * _NS
_CH = 128

_LAYERS = (
    'dense_item_self_biinter', 'dense_item_self_siinter',
    'dense_item_onehop_biinter', 'dense_item_onehop_siinter',
    'dense_user_self_biinter', 'dense_user_self_siinter',
    'dense_user_onehop_biinter', 'dense_user_onehop_siinter',
    'dense_item_cate_self', 'dense_item_cate_hop1',
    'dense_user_cate_self', 'dense_user_cate_hop1',
    'dense_item_addgate', 'dense_item_erasegate',
    'dense_user_addgate', 'dense_user_erasegate',
    'FC_pre',
)


def _sc_gather(dwsc_tab, emb_tab, user_tab, ubias, ibias,
               dwsc_idx, emb_idx, uidx, iidx):
    r_dwsc = dwsc_idx.shape[0]
    r_emb = emb_idx.shape[0]
    b = uidx.shape[0]
    e = emb_tab.shape[1]
    n_dwsc = r_dwsc // (_NW * _CH)
    n_emb = r_emb // (_NW * _CH)
    n_u = b // (_NW * _CH)

    mesh = plsc.VectorSubcoreMesh(core_axis_name="c", subcore_axis_name="s",
                                  num_cores=_NC, num_subcores=_NS)

    @functools.partial(
        pl.kernel,
        out_type=[
            jax.ShapeDtypeStruct((r_dwsc, e), jnp.float32),
            jax.ShapeDtypeStruct((r_emb, e), jnp.float32),
            jax.ShapeDtypeStruct((b, e), jnp.float32),
            jax.ShapeDtypeStruct((b,), jnp.float32),
            jax.ShapeDtypeStruct((b,), jnp.float32),
        ],
        mesh=mesh,
        compiler_params=pltpu.CompilerParams(use_tc_tiling_on_sc=False),
        scratch_types=[
            pltpu.VMEM((_CH,), jnp.int32),
            pltpu.VMEM((_CH, e), jnp.float32),
            pltpu.VMEM((_CH,), jnp.float32),
            pltpu.SemaphoreType.DMA,
        ],
    )
    def run(dwsc_ref, emb_ref, user_ref, ub_ref, ib_ref,
            dwsc_i_ref, emb_i_ref, u_i_ref, i_i_ref,
            dwsc_o, emb_o, urow_o, ubg_o, ibg_o,
            idx_v, rows_v, val_v, sem):
        wid = lax.axis_index("s") * _NC + lax.axis_index("c")

        def row_gather(tab, idx_hbm, out_hbm, nchunk):
            base0 = wid * (nchunk * _CH)

            def body(i, carry):
                off = base0 + i * _CH
                pltpu.sync_copy(idx_hbm.at[pl.ds(off, _CH)], idx_v)
                pltpu.async_copy(tab.at[idx_v], rows_v, sem).wait()
                pltpu.sync_copy(rows_v, out_hbm.at[pl.ds(off, _CH)])
                return carry

            lax.fori_loop(0, nchunk, body, 0)

        row_gather(dwsc_ref, dwsc_i_ref, dwsc_o, n_dwsc)
        row_gather(emb_ref, emb_i_ref, emb_o, n_emb)
        row_gather(user_ref, u_i_ref, urow_o, n_u)

        def elem_gather(tab, idx_hbm, out_hbm):
            off = wid * _CH
            pltpu.sync_copy(idx_hbm.at[pl.ds(off, _CH)], idx_v)
            pltpu.async_copy(tab.at[idx_v], val_v, sem).wait()
            pltpu.sync_copy(val_v, out_hbm.at[pl.ds(off, _CH)])

        elem_gather(ub_ref, u_i_ref, ubg_o)
        elem_gather(ib_ref, i_i_ref, ibg_o)

    return run(dwsc_tab, emb_tab, user_tab, ubias, ibias,
               dwsc_idx, emb_idx, uidx, iidx)


def _tc_forward(u_rows, dwsc_rows, emb_rows, ub_g, ib_g,
                isc, ioc, usc, uoc, genre, gender, age, occ, miu2,
                wb, B, NB, E):
    BB = 128
    G = B // BB
    P = BB * NB
    off_ds = (B * NB * 8) // (BB * 8)
    off_unb = (B * NB) // P
    off_ie = (2 * B * NB) // BB

    def body(u_ref, dh_ref, ds_ref, inb_ref, unb_ref, ie_ref,
             ub_ref, ib_ref, isc_ref, ioc_ref, usc_ref, uoc_ref,
             g_ref, gen_ref, age_ref, occ_ref, miu_ref, *rest):
        out_ref = rest[-1]
        wrefs = rest[:-1]
        wd = {}
        for k, name in enumerate(_LAYERS):
            wd[name] = (wrefs[2 * k][...], wrefs[2 * k + 1][...])

        def dot(x, w):
            return jax.lax.dot_general(
                x, w, (((1,), (0,)), ((), ())),
                precision=jax.lax.Precision.HIGHEST,
                preferred_element_type=jnp.float32)

        def lin(x, name):
            w, bb = wd[name]
            return dot(x, w) + bb

        def lin2(x1, x2, name):
            w, bb = wd[name]
            return dot(x1, w[:E, :]) + dot(x2, w[E:, :]) + bb

        def leaky(x):
            return jnp.where(x >= 0, x, 0.01 * x)

        def sig(x):
            return 1.0 / (1.0 + jnp.exp(-x))

        def interact(S, Q, bi, si):
            deep = 0.5 * (S * S - Q)
            return leaky(lin(deep, bi)) + leaky(lin(S, si))

        def sum8(v, npair):
            r = v.reshape(npair, 8, E)
            return r.sum(axis=1), (r * r).sum(axis=1)

        def onehot_counts(idx, V):
            n, k = idx.shape
            io = lax.broadcasted_iota(jnp.int32, (n, V), 1)
            c = jnp.zeros((n, V), jnp.float32)
            for s in range(k):
                c = c + (idx[:, s:s + 1] == io).astype(jnp.float32)
            return c

        g = g_ref[...]
        g2 = g * g
        NG = g.shape[0]

        Sd, Qd = sum8(ds_ref[...], BB)
        cg = onehot_counts(isc_ref[...], NG)
        att1 = interact(Sd + dot(cg, g), Qd + dot(cg, g2),
                        'dense_item_self_biinter', 'dense_item_self_siinter')
        item_cat = jnp.maximum(lin2(ie_ref[...], att1, 'dense_item_cate_self'), 0.0)

        Sh, Qh = sum8(dh_ref[...], P)
        chg = onehot_counts(ioc_ref[...], NG)
        att2 = interact(Sh + dot(chg, g), Qh + dot(chg, g2),
                        'dense_item_onehop_biinter', 'dense_item_onehop_siinter')
        item_nb = jnp.maximum(lin2(inb_ref[...], att2, 'dense_item_cate_hop1'), 0.0)
        item_nb_agg = item_nb.reshape(BB, NB, E).sum(axis=1) * (1.0 / NB)

        add_g = sig(lin2(item_cat, item_nb_agg, 'dense_item_addgate'))
        erase_g = sig(lin2(item_cat, item_nb_agg, 'dense_item_erasegate'))
        item_final = item_cat * (1.0 - erase_g) + item_nb_agg * add_g

        usc_v = usc_ref[...]

        def one_tab(idxcol, tab):
            V = tab.shape[0]
            io = lax.broadcasted_iota(jnp.int32, (BB, V), 1)
            oh = (idxcol == io).astype(jnp.float32)
            return dot(oh, tab), dot(oh, tab * tab)

        Sg, Qg = one_tab(usc_v[:, 0:1], gen_ref[...])
        Sa, Qa = one_tab(usc_v[:, 1:2], age_ref[...])
        So, Qo = one_tab(usc_v[:, 2:3], occ_ref[...])
        att3 = interact(Sg + Sa + So, Qg + Qa + Qo,
                        'dense_user_self_biinter', 'dense_user_self_siinter')
        user_cat = jnp.maximum(lin2(u_ref[...], att3, 'dense_user_cate_self'), 0.0)

        cu = onehot_counts(uoc_ref[...], NG)
        att4 = interact(dot(cu, g), dot(cu, g2),
                        'dense_user_onehop_biinter', 'dense_user_onehop_siinter')
        user_nb = jnp.maximum(lin2(unb_ref[...], att4, 'dense_user_cate_hop1'), 0.0)
        user_nb_agg = user_nb.reshape(BB, NB, E).sum(axis=1) * (1.0 / NB)

        uadd = sig(lin2(user_cat, user_nb_agg, 'dense_user_addgate'))
        uerase = sig(lin2(user_cat, user_nb_agg, 'dense_user_erasegate'))
        user_final = user_cat * (1.0 - uerase) + user_nb_agg * uadd

        wfc, bfc = wd['FC_pre']
        pred = ((user_final * wfc[:, :E]).sum(axis=1)
                + (item_final * wfc[:, E:]).sum(axis=1))
        pred = pred + bfc[0, 0] + miu_ref[0, 0] + ub_ref[...] + ib_ref[...]
        out_ref[...] = pred

    def blk(shape, row_off=0):
        return pl.BlockSpec(shape, lambda i, o=row_off: (o + i, 0))

    def blk1(n):
        return pl.BlockSpec((n,), lambda i: (i,))

    def full(a):
        return pl.BlockSpec(a.shape, lambda i: (0, 0))

    in_specs = [
        blk((BB, E)),
        blk((P * 8, E)),
        blk((BB * 8, E), off_ds),
        blk((P, E)),
        blk((P, E), off_unb),
        blk((BB, E), off_ie),
        blk1(BB), blk1(BB),
        blk((BB, 3)), blk((P, 3)), blk((BB, 3)), blk((P, 3)),
        full(genre), full(gender), full(age), full(occ), full(miu2),
    ]
    ops = [u_rows, dwsc_rows, dwsc_rows, emb_rows, emb_rows, emb_rows,
           ub_g, ib_g, isc, ioc, usc, uoc, genre, gender, age, occ, miu2]
    for w, bb_ in wb:
        in_specs.append(full(w))
        in_specs.append(full(bb_))
        ops.append(w)
        ops.append(bb_)

    out = pl.pallas_call(
        body,
        grid=(G,),
        in_specs=in_specs,
        out_specs=blk1(BB),
        out_shape=jax.ShapeDtypeStruct((B,), jnp.float32),
    )(*ops)
    return out


def kernel(params, user, item, user_self_cate, user_onehop_id, user_onehop_cate,
           item_self_cate, item_self_director, item_self_writer, item_self_star,
           item_self_country, item_onehop_id, item_onehop_cate,
           item_onehop_director, item_onehop_writer, item_onehop_star,
           item_onehop_country):
    p = params
    B = user.shape[0]
    NB = user_onehop_id.shape[1]
    E = p['user_embed'].shape[1]

    d_tab = p['director_embed']
    w_tab = p['writer_embed']
    s_tab = p['star_embed']
    c_tab = p['country_embed']
    off_w = d_tab.shape[0]
    off_s = off_w + w_tab.shape[0]
    off_c = off_s + s_tab.shape[0]
    dwsc_tab = jnp.concatenate([d_tab, w_tab, s_tab, c_tab], axis=0)

    hop8 = jnp.concatenate(
        [item_onehop_director, item_onehop_writer + off_w,
         item_onehop_star + off_s, item_onehop_country + off_c], axis=2)
    self8 = jnp.concatenate(
        [item_self_director, item_self_writer + off_w,
         item_self_star + off_s, item_self_country + off_c], axis=1)
    dwsc_idx = jnp.concatenate([hop8.reshape(-1), self8.reshape(-1)])
    emb_idx = jnp.concatenate(
        [item_onehop_id.reshape(-1), user_onehop_id.reshape(-1), item])

    dwsc_rows, emb_rows, u_rows, ub_g, ib_g = _sc_gather(
        dwsc_tab, p['item_embed'], p['user_embed'],
        p['user_bias'].reshape(-1), p['item_bias'].reshape(-1),
        dwsc_idx, emb_idx, user, item)

    wb = []
    for name in _LAYERS[:-1]:
        wb.append((p[name + '_w'], p[name + '_b'].reshape(1, -1)))
    wb.append((p['FC_pre_w'].reshape(1, 2 * E), p['FC_pre_b'].reshape(1, 1)))

    return _tc_forward(
        u_rows, dwsc_rows, emb_rows, ub_g, ib_g,
        item_self_cate, item_onehop_cate.reshape(B * NB, 3),
        user_self_cate, user_onehop_cate.reshape(B * NB, 3),
        p['genre_embed'], p['gender_embed'], p['age_embed'],
        p['occupation_embed'], p['miu'].reshape(1, 1),
        wb, B, NB, E)


# speedup vs baseline: 2.6866x; 1.1072x over previous
import functools

import jax
import jax.numpy as jnp
from jax import lax
from jax.experimental import pallas as pl
from jax.experimental.pallas import tpu as pltpu
from jax.experimental.pallas import tpu_sc as plsc

_NC, _NS = 2, 16
_NW = _NC * _NS
_CH = 128

_LAYERS = (
    'dense_item_self_biinter', 'dense_item_self_siinter',
    'dense_item_onehop_biinter', 'dense_item_onehop_siinter',
    'dense_user_self_biinter', 'dense_user_self_siinter',
    'dense_user_onehop_biinter', 'dense_user_onehop_siinter',
    'dense_item_cate_self', 'dense_item_cate_hop1',
    'dense_user_cate_self', 'dense_user_cate_hop1',
    'dense_item_addgate', 'dense_item_erasegate',
    'dense_user_addgate', 'dense_user_erasegate',
    'FC_pre',
)


def _sc_gather(dwsc_tab, emb_tab, user_tab, ubias, ibias,
               hop_idx, self_idx, ih_idx, uh_idx, iidx, uidx):
    e = emb_tab.shape[1]
    b = uidx.shape[0]

    row_jobs = [
        (0, hop_idx), (0, self_idx),
        (1, ih_idx), (1, uh_idx), (1, iidx), (2, uidx),
    ]
    nchunks = [ix.shape[0] // (_NW * _CH) for _, ix in row_jobs]

    out_type = [jax.ShapeDtypeStruct((ix.shape[0], e), jnp.float32)
                for _, ix in row_jobs]
    out_type += [jax.ShapeDtypeStruct((b,), jnp.float32),
                 jax.ShapeDtypeStruct((b,), jnp.float32)]

    mesh = plsc.VectorSubcoreMesh(core_axis_name="c", subcore_axis_name="s",
                                  num_cores=_NC, num_subcores=_NS)

    @functools.partial(
        pl.kernel,
        out_type=out_type,
        mesh=mesh,
        compiler_params=pltpu.CompilerParams(use_tc_tiling_on_sc=False),
        scratch_types=[
            pltpu.VMEM((_CH,), jnp.int32),
            pltpu.VMEM((_CH, e), jnp.float32),
            pltpu.VMEM((_CH,), jnp.float32),
            pltpu.SemaphoreType.DMA,
        ],
    )
    def run(*refs):
        tab_refs = refs[0:3]
        ub_r, ib_r = refs[3], refs[4]
        idx_refs = refs[5:11]
        out_refs = refs[11:17]
        ubg_o, ibg_o = refs[17], refs[18]
        idx_v, rows_v, val_v, sem = refs[19:23]

        wid = lax.axis_index("s") * _NC + lax.axis_index("c")

        def row_gather(tab, idx_hbm, out_hbm, nchunk):
            base0 = wid * (nchunk * _CH)

            def body(i, carry):
                off = base0 + i * _CH
                pltpu.sync_copy(idx_hbm.at[pl.ds(off, _CH)], idx_v)
                pltpu.async_copy(tab.at[idx_v], rows_v, sem).wait()
                pltpu.sync_copy(rows_v, out_hbm.at[pl.ds(off, _CH)])
                return carry

            lax.fori_loop(0, nchunk, body, 0)

        for (ti, _), ix_ref, o_ref, nc in zip(row_jobs, idx_refs, out_refs,
                                              nchunks):
            row_gather(tab_refs[ti], ix_ref, o_ref, nc)

        def elem_gather(tab, idx_hbm, out_hbm):
            off = wid * _CH
            pltpu.sync_copy(idx_hbm.at[pl.ds(off, _CH)], idx_v)
            pltpu.async_copy(tab.at[idx_v], val_v, sem).wait()
            pltpu.sync_copy(val_v, out_hbm.at[pl.ds(off, _CH)])

        elem_gather(ub_r, idx_refs[5], ubg_o)
        elem_gather(ib_r, idx_refs[4], ibg_o)

    return run(dwsc_tab, emb_tab, user_tab, ubias, ibias,
               hop_idx, self_idx, ih_idx, uh_idx, iidx, uidx)


def _tc_forward(u_rows, ih3, uh3, ie, dh4, ds3, ub_g, ib_g,
                isc, ioc_t, usc, uoc_t, genre, gender, age, occ, miu2,
                wb, B, NB, E):
    BB = 128
    G = B // BB
    P = BB * NB

    def body(u_ref, ih_ref, uh_ref, ie_ref, dh_ref, ds_ref, ub_ref, ib_ref,
             isc_ref, ioc_ref, usc_ref, uoc_ref,
             g_ref, gen_ref, age_ref, occ_ref, miu_ref, *rest):
        out_ref = rest[-1]
        wrefs = rest[:-1]
        wd = {}
        for k, name in enumerate(_LAYERS):
            wd[name] = (wrefs[2 * k][...], wrefs[2 * k + 1][...])

        def dot(x, w):
            return jax.lax.dot_general(
                x, w, (((1,), (0,)), ((), ())),
                precision=jax.lax.Precision.HIGHEST,
                preferred_element_type=jnp.float32)

        def lin(x, name):
            w, bb = wd[name]
            return dot(x, w) + bb

        def lin2(x1, x2, name):
            w, bb = wd[name]
            return dot(x1, w[:E, :]) + dot(x2, w[E:, :]) + bb

        def leaky(x):
            return jnp.where(x >= 0, x, 0.01 * x)

        def sig(x):
            return 1.0 / (1.0 + jnp.exp(-x))

        def interact(S, Q, bi, si):
            deep = 0.5 * (S * S - Q)
            return leaky(lin(deep, bi)) + leaky(lin(S, si))

        def sumsq0(v):
            return v.sum(axis=0), (v * v).sum(axis=0)

        def onehot_counts(idx, V):
            n, k = idx.shape
            io = lax.broadcasted_iota(jnp.int32, (n, V), 1)
            c = jnp.zeros((n, V), jnp.float32)
            for s in range(k):
                c = c + (idx[:, s:s + 1] == io).astype(jnp.float32)
            return c

        g = g_ref[...]
        g2 = g * g
        NG = g.shape[0]

        Sd, Qd = sumsq0(ds_ref[...])
        cg = onehot_counts(isc_ref[...], NG)
        att1 = interact(Sd + dot(cg, g), Qd + dot(cg, g2),
                        'dense_item_self_biinter', 'dense_item_self_siinter')
        item_cat = jnp.maximum(lin2(ie_ref[...], att1, 'dense_item_cate_self'), 0.0)

        Sh, Qh = sumsq0(dh_ref[...].reshape(8, P, E))
        chg = onehot_counts(ioc_ref[...].reshape(P, 3), NG)
        att2 = interact(Sh + dot(chg, g), Qh + dot(chg, g2),
                        'dense_item_onehop_biinter', 'dense_item_onehop_siinter')
        inb = ih_ref[...].reshape(P, E)
        item_nb = jnp.maximum(lin2(inb, att2, 'dense_item_cate_hop1'), 0.0)
        item_nb_agg = item_nb.reshape(NB, BB, E).sum(axis=0) * (1.0 / NB)

        add_g = sig(lin2(item_cat, item_nb_agg, 'dense_item_addgate'))
        erase_g = sig(lin2(item_cat, item_nb_agg, 'dense_item_erasegate'))
        item_final = item_cat * (1.0 - erase_g) + item_nb_agg * add_g

        usc_v = usc_ref[...]

        def one_tab(idxcol, tab):
            V = tab.shape[0]
            io = lax.broadcasted_iota(jnp.int32, (BB, V), 1)
            oh = (idxcol == io).astype(jnp.float32)
            return dot(oh, tab), dot(oh, tab * tab)

        Sg, Qg = one_tab(usc_v[:, 0:1], gen_ref[...])
        Sa, Qa = one_tab(usc_v[:, 1:2], age_ref[...])
        So, Qo = one_tab(usc_v[:, 2:3], occ_ref[...])
        att3 = interact(Sg + Sa + So, Qg + Qa + Qo,
                        'dense_user_self_biinter', 'dense_user_self_siinter')
        user_cat = jnp.maximum(lin2(u_ref[...], att3, 'dense_user_cate_self'), 0.0)

        cu = onehot_counts(uoc_ref[...].reshape(P, 3), NG)
        att4 = interact(dot(cu, g), dot(cu, g2),
                        'dense_user_onehop_biinter', 'dense_user_onehop_siinter')
        unb = uh_ref[...].reshape(P, E)
        user_nb = jnp.maximum(lin2(unb, att4, 'dense_user_cate_hop1'), 0.0)
        user_nb_agg = user_nb.reshape(NB, BB, E).sum(axis=0) * (1.0 / NB)

        uadd = sig(lin2(user_cat, user_nb_agg, 'dense_user_addgate'))
        uerase = sig(lin2(user_cat, user_nb_agg, 'dense_user_erasegate'))
        user_final = user_cat * (1.0 - uerase) + user_nb_agg * uadd

        wfc, bfc = wd['FC_pre']
        pred = ((user_final * wfc[:, :E]).sum(axis=1)
                + (item_final * wfc[:, E:]).sum(axis=1))
        pred = pred + bfc[0, 0] + miu_ref[0, 0] + ub_ref[...] + ib_ref[...]
        out_ref[...] = pred

    def blk1(n):
        return pl.BlockSpec((n,), lambda i: (i,))

    def full(a):
        return pl.BlockSpec(a.shape, lambda i: (0, 0))

    in_specs = [
        pl.BlockSpec((BB, E), lambda i: (i, 0)),
        pl.BlockSpec((NB, BB, E), lambda i: (0, i, 0)),
        pl.BlockSpec((NB, BB, E), lambda i: (0, i, 0)),
        pl.BlockSpec((BB, E), lambda i: (i, 0)),
        pl.BlockSpec((8, NB, BB, E), lambda i: (0, 0, i, 0)),
        pl.BlockSpec((8, BB, E), lambda i: (0, i, 0)),
        blk1(BB), blk1(BB),
        pl.BlockSpec((BB, 3), lambda i: (i, 0)),
        pl.BlockSpec((NB, BB, 3), lambda i: (0, i, 0)),
        pl.BlockSpec((BB, 3), lambda i: (i, 0)),
        pl.BlockSpec((NB, BB, 3), lambda i: (0, i, 0)),
        full(genre), full(gender), full(age), full(occ), full(miu2),
    ]
    ops = [u_rows, ih3, uh3, ie, dh4, ds3, ub_g, ib_g,
           isc, ioc_t, usc, uoc_t, genre, gender, age, occ, miu2]
    for w, bb_ in wb:
        in_specs.append(full(w))
        in_specs.append(full(bb_))
        ops.append(w)
        ops.append(bb_)

    out = pl.pallas_call(
        body,
        grid=(G,),
        in_specs=in_specs,
        out_specs=blk1(BB),
        out_shape=jax.ShapeDtypeStruct((B,), jnp.float32),
    )(*ops)
    return out


def kernel(params, user, item, user_self_cate, user_onehop_id, user_onehop_cate,
           item_self_cate, item_self_director, item_self_writer, item_self_star,
           item_self_country, item_onehop_id, item_onehop_cate,
           item_onehop_director, item_onehop_writer, item_onehop_star,
           item_onehop_country):
    p = params
    B = user.shape[0]
    NB = user_onehop_id.shape[1]
    E = p['user_embed'].shape[1]

    d_tab = p['director_embed']
    w_tab = p['writer_embed']
    s_tab = p['star_embed']
    c_tab = p['country_embed']
    off_w = d_tab.shape[0]
    off_s = off_w + w_tab.shape[0]
    off_c = off_s + s_tab.shape[0]
    dwsc_tab = jnp.concatenate([d_tab, w_tab, s_tab, c_tab], axis=0)

    def tslot(a, k, off):
        return (a[:, :, k] + off).transpose(1, 0).reshape(-1)

    hop_idx = jnp.concatenate([
        tslot(item_onehop_director, 0, 0), tslot(item_onehop_director, 1, 0),
        tslot(item_onehop_writer, 0, off_w), tslot(item_onehop_writer, 1, off_w),
        tslot(item_onehop_star, 0, off_s), tslot(item_onehop_star, 1, off_s),
        tslot(item_onehop_star, 2, off_s),
        tslot(item_onehop_country, 0, off_c)])
    self_idx = jnp.concatenate([
        item_self_director[:, 0], item_self_director[:, 1],
        item_self_writer[:, 0] + off_w, item_self_writer[:, 1] + off_w,
        item_self_star[:, 0] + off_s, item_self_star[:, 1] + off_s,
        item_self_star[:, 2] + off_s,
        item_self_country[:, 0] + off_c])
    ih_idx = item_onehop_id.transpose(1, 0).reshape(-1)
    uh_idx = user_onehop_id.transpose(1, 0).reshape(-1)

    (hop_rows, self_rows, ih_rows, uh_rows, ie, u_rows,
     ub_g, ib_g) = _sc_gather(
        dwsc_tab, p['item_embed'], p['user_embed'],
        p['user_bias'].reshape(-1), p['item_bias'].reshape(-1),
        hop_idx, self_idx, ih_idx, uh_idx, item, user)

    dh4 = hop_rows.reshape(8, NB, B, E)
    ds3 = self_rows.reshape(8, B, E)
    ih3 = ih_rows.reshape(NB, B, E)
    uh3 = uh_rows.reshape(NB, B, E)

    wb = []
    for name in _LAYERS[:-1]:
        wb.append((p[name + '_w'], p[name + '_b'].reshape(1, -1)))
    wb.append((p['FC_pre_w'].reshape(1, 2 * E), p['FC_pre_b'].reshape(1, 1)))

    return _tc_forward(
        u_rows, ih3, uh3, ie, dh4, ds3, ub_g, ib_g,
        item_self_cate, item_onehop_cate.transpose(1, 0, 2),
        user_self_cate, user_onehop_cate.transpose(1, 0, 2),
        p['genre_embed'], p['gender_embed'], p['age_embed'],
        p['occupation_embed'], p['miu'].reshape(1, 1),
        wb, B, NB, E)


# speedup vs baseline: 3.5756x; 1.3309x over previous
import functools

import jax
import jax.numpy as jnp
from jax import lax
from jax.experimental import pallas as pl
from jax.experimental.pallas import tpu as pltpu
from jax.experimental.pallas import tpu_sc as plsc

_NC, _NS = 2, 16
_NW = _NC * _NS
_CH = 128

_LAYERS = (
    'dense_item_self_biinter', 'dense_item_self_siinter',
    'dense_item_onehop_biinter', 'dense_item_onehop_siinter',
    'dense_user_self_biinter', 'dense_user_self_siinter',
    'dense_user_onehop_biinter', 'dense_user_onehop_siinter',
    'dense_item_cate_self', 'dense_item_cate_hop1',
    'dense_user_cate_self', 'dense_user_cate_hop1',
    'dense_item_addgate', 'dense_item_erasegate',
    'dense_user_addgate', 'dense_user_erasegate',
    'FC_pre',
)


def _sc_gather(dwsc_tab, emb_tab, user_tab, ubias, ibias,
               hop_idx, self_idx, ih_idx, uh_idx, iidx, uidx):
    e = emb_tab.shape[1]
    b = uidx.shape[0]

    row_jobs = [
        (0, hop_idx), (0, self_idx),
        (1, ih_idx), (1, uh_idx), (1, iidx), (2, uidx),
    ]
    nchunks = [ix.shape[0] // (_NW * _CH) for _, ix in row_jobs]

    out_type = [jax.ShapeDtypeStruct((ix.shape[0], e), jnp.float32)
                for _, ix in row_jobs]
    out_type += [jax.ShapeDtypeStruct((b,), jnp.float32),
                 jax.ShapeDtypeStruct((b,), jnp.float32)]

    mesh = plsc.VectorSubcoreMesh(core_axis_name="c", subcore_axis_name="s",
                                  num_cores=_NC, num_subcores=_NS)

    @functools.partial(
        pl.kernel,
        out_type=out_type,
        mesh=mesh,
        compiler_params=pltpu.CompilerParams(use_tc_tiling_on_sc=False),
        scratch_types=[
            pltpu.VMEM((_CH,), jnp.int32),
            pltpu.VMEM((_CH, e), jnp.float32),
            pltpu.VMEM((_CH,), jnp.float32),
            pltpu.SemaphoreType.DMA,
        ],
    )
    def run(*refs):
        tab_refs = refs[0:3]
        ub_r, ib_r = refs[3], refs[4]
        idx_refs = refs[5:11]
        out_refs = refs[11:17]
        ubg_o, ibg_o = refs[17], refs[18]
        idx_v, rows_v, val_v, sem = refs[19:23]

        wid = lax.axis_index("s") * _NC + lax.axis_index("c")

        def row_gather(tab, idx_hbm, out_hbm, nchunk):
            base0 = wid * (nchunk * _CH)

            def body(i, carry):
                off = base0 + i * _CH
                pltpu.sync_copy(idx_hbm.at[pl.ds(off, _CH)], idx_v)
                pltpu.async_copy(tab.at[idx_v], rows_v, sem).wait()
                pltpu.sync_copy(rows_v, out_hbm.at[pl.ds(off, _CH)])
                return carry

            lax.fori_loop(0, nchunk, body, 0)

        for (ti, _), ix_ref, o_ref, nc in zip(row_jobs, idx_refs, out_refs,
                                              nchunks):
            row_gather(tab_refs[ti], ix_ref, o_ref, nc)

        def elem_gather(tab, idx_hbm, out_hbm):
            off = wid * _CH
            pltpu.sync_copy(idx_hbm.at[pl.ds(off, _CH)], idx_v)
            pltpu.async_copy(tab.at[idx_v], val_v, sem).wait()
            pltpu.sync_copy(val_v, out_hbm.at[pl.ds(off, _CH)])

        elem_gather(ub_r, idx_refs[5], ubg_o)
        elem_gather(ib_r, idx_refs[4], ibg_o)

    return run(dwsc_tab, emb_tab, user_tab, ubias, ibias,
               hop_idx, self_idx, ih_idx, uh_idx, iidx, uidx)


def _tc_forward(u_rows, ih3, uh3, ie, dh4, ds3, ub2, ib2,
                isc6, ioc6, usc6, uoc6, genre, gender, age, occ, miu2,
                wb, B, NB, E):
    BB = 128
    H = BB // 2
    G = B // BB
    PH = H * NB

    def body(u_ref, ih_ref, uh_ref, ie_ref, dh_ref, ds_ref, ub_ref, ib_ref,
             isc_ref, ioc_ref, usc_ref, uoc_ref,
             g_ref, gen_ref, age_ref, occ_ref, miu_ref, *rest):
        out_ref = rest[-1]
        wrefs = rest[:-1]
        wd = {}
        for k, name in enumerate(_LAYERS):
            wd[name] = (wrefs[2 * k][...], wrefs[2 * k + 1][...])

        def dot(x, w):
            return jax.lax.dot_general(
                x, w, (((1,), (0,)), ((), ())),
                precision=jax.lax.Precision.HIGHEST,
                preferred_element_type=jnp.float32)

        def bd(w):
            k, n = w.shape
            z = jnp.zeros((k, n), jnp.float32)
            return jnp.concatenate(
                [jnp.concatenate([w, z], axis=1),
                 jnp.concatenate([z, w], axis=1)], axis=0)

        def lin(x, name):
            w, bb = wd[name]
            return dot(x, bd(w)) + jnp.concatenate([bb, bb], axis=1)

        def lin2(x1, x2, name):
            w, bb = wd[name]
            return (dot(x1, bd(w[:E, :])) + dot(x2, bd(w[E:, :]))
                    + jnp.concatenate([bb, bb], axis=1))

        def leaky(x):
            return jnp.where(x >= 0, x, 0.01 * x)

        def sig(x):
            return 1.0 / (1.0 + jnp.exp(-x))

        def interact(S, Q, bi, si):
            deep = 0.5 * (S * S - Q)
            return leaky(lin(deep, bi)) + leaky(lin(S, si))

        def sumsq0(v):
            return v.sum(axis=0), (v * v).sum(axis=0)

        def counts_packed(idx2k, V):
            n, k2 = idx2k.shape
            k = k2 // 2
            col = lax.broadcasted_iota(jnp.int32, (n, 2 * V), 1)
            colv = jnp.where(col < V, col, col - V)
            even = col < V
            c = jnp.zeros((n, 2 * V), jnp.float32)
            one = jnp.float32(1.0)
            zero = jnp.float32(0.0)
            for s in range(k):
                c = c + jnp.where(even & (idx2k[:, s:s + 1] == colv), one, zero)
                c = c + jnp.where((~even) & (idx2k[:, k + s:k + s + 1] == colv),
                                  one, zero)
            return c

        g = g_ref[...]
        g2 = g * g
        NG = g.shape[0]

        Sd, Qd = sumsq0(ds_ref[...])
        cg = counts_packed(isc_ref[...], NG)
        att1 = interact(Sd + dot(cg, bd(g)), Qd + dot(cg, bd(g2)),
                        'dense_item_self_biinter', 'dense_item_self_siinter')
        item_cat = jnp.maximum(lin2(ie_ref[...], att1, 'dense_item_cate_self'), 0.0)

        Sh, Qh = sumsq0(dh_ref[...].reshape(8, PH, 2 * E))
        chg = counts_packed(ioc_ref[...].reshape(PH, 6), NG)
        att2 = interact(Sh + dot(chg, bd(g)), Qh + dot(chg, bd(g2)),
                        'dense_item_onehop_biinter', 'dense_item_onehop_siinter')
        inb = ih_ref[...].reshape(PH, 2 * E)
        item_nb = jnp.maximum(lin2(inb, att2, 'dense_item_cate_hop1'), 0.0)
        item_nb_agg = item_nb.reshape(NB, H, 2 * E).sum(axis=0) * (1.0 / NB)

        add_g = sig(lin2(item_cat, item_nb_agg, 'dense_item_addgate'))
        erase_g = sig(lin2(item_cat, item_nb_agg, 'dense_item_erasegate'))
        item_final = item_cat * (1.0 - erase_g) + item_nb_agg * add_g

        usc_v = usc_ref[...]

        def one_tab(s, tab):
            V = tab.shape[0]
            idx2 = jnp.concatenate(
                [usc_v[:, s:s + 1], usc_v[:, 3 + s:4 + s]], axis=1)
            cp = counts_packed(idx2, V)
            return dot(cp, bd(tab)), dot(cp, bd(tab * tab))

        Sg, Qg = one_tab(0, gen_ref[...])
        Sa, Qa = one_tab(1, age_ref[...])
        So, Qo = one_tab(2, occ_ref[...])
        att3 = interact(Sg + Sa + So, Qg + Qa + Qo,
                        'dense_user_self_biinter', 'dense_user_self_siinter')
        user_cat = jnp.maximum(lin2(u_ref[...], att3, 'dense_user_cate_self'), 0.0)

        cu = counts_packed(uoc_ref[...].reshape(PH, 6), NG)
        att4 = interact(dot(cu, bd(g)), dot(cu, bd(g2)),
                        'dense_user_onehop_biinter', 'dense_user_onehop_siinter')
        unb = uh_ref[...].reshape(PH, 2 * E)
        user_nb = jnp.maximum(lin2(unb, att4, 'dense_user_cate_hop1'), 0.0)
        user_nb_agg = user_nb.reshape(NB, H, 2 * E).sum(axis=0) * (1.0 / NB)

        uadd = sig(lin2(user_cat, user_nb_agg, 'dense_user_addgate'))
        uerase = sig(lin2(user_cat, user_nb_agg, 'dense_user_erasegate'))
        user_final = user_cat * (1.0 - uerase) + user_nb_agg * uadd

        wfc, bfc = wd['FC_pre']
        wu = wfc[:, :E]
        wi = wfc[:, E:]
        const = bfc[0, 0] + miu_ref[0, 0]
        pe = ((user_final[:, :E] * wu).sum(axis=1)
              + (item_final[:, :E] * wi).sum(axis=1)
              + ub_ref[:, 0] + ib_ref[:, 0] + const)
        po = ((user_final[:, E:] * wu).sum(axis=1)
              + (item_final[:, E:] * wi).sum(axis=1)
              + ub_ref[:, 1] + ib_ref[:, 1] + const)
        out_ref[...] = jnp.stack([pe, po], axis=1)

    def full(a):
        return pl.BlockSpec(a.shape, lambda i: (0, 0))

    in_specs = [
        pl.BlockSpec((H, 2 * E), lambda i: (i, 0)),
        pl.BlockSpec((NB, H, 2 * E), lambda i: (0, i, 0)),
        pl.BlockSpec((NB, H, 2 * E), lambda i: (0, i, 0)),
        pl.BlockSpec((H, 2 * E), lambda i: (i, 0)),
        pl.BlockSpec((8, NB, H, 2 * E), lambda i: (0, 0, i, 0)),
        pl.BlockSpec((8, H, 2 * E), lambda i: (0, i, 0)),
        pl.BlockSpec((H, 2), lambda i: (i, 0)),
        pl.BlockSpec((H, 2), lambda i: (i, 0)),
        pl.BlockSpec((H, 6), lambda i: (i, 0)),
        pl.BlockSpec((NB, H, 6), lambda i: (0, i, 0)),
        pl.BlockSpec((H, 6), lambda i: (i, 0)),
        pl.BlockSpec((NB, H, 6), lambda i: (0, i, 0)),
        full(genre), full(gender), full(age), full(occ), full(miu2),
    ]
    ops = [u_rows, ih3, uh3, ie, dh4, ds3, ub2, ib2,
           isc6, ioc6, usc6, uoc6, genre, gender, age, occ, miu2]
    for w, bb_ in wb:
        in_specs.append(full(w))
        in_specs.append(full(bb_))
        ops.append(w)
        ops.append(bb_)

    out = pl.pallas_call(
        body,
        grid=(G,),
        in_specs=in_specs,
        out_specs=pl.BlockSpec((H, 2), lambda i: (i, 0)),
        out_shape=jax.ShapeDtypeStruct((B // 2, 2), jnp.float32),
    )(*ops)
    return out.reshape(B)


def kernel(params, user, item, user_self_cate, user_onehop_id, user_onehop_cate,
           item_self_cate, item_self_director, item_self_writer, item_self_star,
           item_self_country, item_onehop_id, item_onehop_cate,
           item_onehop_director, item_onehop_writer, item_onehop_star,
           item_onehop_country):
    p = params
    B = user.shape[0]
    NB = user_onehop_id.shape[1]
    E = p['user_embed'].shape[1]

    d_tab = p['director_embed']
    w_tab = p['writer_embed']
    s_tab = p['star_embed']
    c_tab = p['country_embed']
    off_w = d_tab.shape[0]
    off_s = off_w + w_tab.shape[0]
    off_c = off_s + s_tab.shape[0]
    dwsc_tab = jnp.concatenate([d_tab, w_tab, s_tab, c_tab], axis=0)

    def tslot(a, k, off):
        return (a[:, :, k] + off).transpose(1, 0).reshape(-1)

    hop_idx = jnp.concatenate([
        tslot(item_onehop_director, 0, 0), tslot(item_onehop_director, 1, 0),
        tslot(item_onehop_writer, 0, off_w), tslot(item_onehop_writer, 1, off_w),
        tslot(item_onehop_star, 0, off_s), tslot(item_onehop_star, 1, off_s),
        tslot(item_onehop_star, 2, off_s),
        tslot(item_onehop_country, 0, off_c)])
    self_idx = jnp.concatenate([
        item_self_director[:, 0], item_self_director[:, 1],
        item_self_writer[:, 0] + off_w, item_self_writer[:, 1] + off_w,
        item_self_star[:, 0] + off_s, item_self_star[:, 1] + off_s,
        item_self_star[:, 2] + off_s,
        item_self_country[:, 0] + off_c])
    ih_idx = item_onehop_id.transpose(1, 0).reshape(-1)
    uh_idx = user_onehop_id.transpose(1, 0).reshape(-1)

    (hop_rows, self_rows, ih_rows, uh_rows, ie, u_rows,
     ub_g, ib_g) = _sc_gather(
        dwsc_tab, p['item_embed'], p['user_embed'],
        p['user_bias'].reshape(-1), p['item_bias'].reshape(-1),
        hop_idx, self_idx, ih_idx, uh_idx, item, user)

    dh4 = hop_rows.reshape(8, NB, B // 2, 2 * E)
    ds3 = self_rows.reshape(8, B // 2, 2 * E)
    ih3 = ih_rows.reshape(NB, B // 2, 2 * E)
    uh3 = uh_rows.reshape(NB, B // 2, 2 * E)
    iep = ie.reshape(B // 2, 2 * E)
    up = u_rows.reshape(B // 2, 2 * E)

    wb = []
    for name in _LAYERS[:-1]:
        wb.append((p[name + '_w'], p[name + '_b'].reshape(1, -1)))
    wb.append((p['FC_pre_w'].reshape(1, 2 * E), p['FC_pre_b'].reshape(1, 1)))

    return _tc_forward(
        up, ih3, uh3, iep, dh4, ds3,
        ub_g.reshape(B // 2, 2), ib_g.reshape(B // 2, 2),
        item_self_cate.reshape(B // 2, 6),
        item_onehop_cate.transpose(1, 0, 2).reshape(NB, B // 2, 6),
        user_self_cate.reshape(B // 2, 6),
        user_onehop_cate.transpose(1, 0, 2).reshape(NB, B // 2, 6),
        p['genre_embed'], p['gender_embed'], p['age_embed'],
        p['occupation_embed'], p['miu'].reshape(1, 1),
        wb, B, NB, E)


# speedup vs baseline: 4.1492x; 1.1604x over previous
import functools

import jax
import jax.numpy as jnp
from jax import lax
from jax.experimental import pallas as pl
from jax.experimental.pallas import tpu as pltpu
from jax.experimental.pallas import tpu_sc as plsc

_NC, _NS = 2, 16
_NW = _NC * _NS
_CH = 128

_LAYERS = (
    'dense_item_self_biinter', 'dense_item_self_siinter',
    'dense_item_onehop_biinter', 'dense_item_onehop_siinter',
    'dense_user_self_biinter', 'dense_user_self_siinter',
    'dense_user_onehop_biinter', 'dense_user_onehop_siinter',
    'dense_item_cate_self', 'dense_item_cate_hop1',
    'dense_user_cate_self', 'dense_user_cate_hop1',
    'dense_item_addgate', 'dense_item_erasegate',
    'dense_user_addgate', 'dense_user_erasegate',
    'FC_pre',
)


def _sc_gather(dwsc_tab, emb_tab, user_tab, ubias, ibias,
               hop_idx, self_idx, ih_idx, uh_idx, iidx, uidx):
    e = emb_tab.shape[1]
    b = uidx.shape[0]

    row_jobs = [
        (0, hop_idx), (0, self_idx),
        (1, ih_idx), (1, uh_idx), (1, iidx), (2, uidx),
    ]
    nchunks = [ix.shape[0] // (_NW * _CH) for _, ix in row_jobs]

    out_type = [jax.ShapeDtypeStruct((ix.shape[0], e), jnp.float32)
                for _, ix in row_jobs]
    out_type += [jax.ShapeDtypeStruct((b,), jnp.float32),
                 jax.ShapeDtypeStruct((b,), jnp.float32)]

    mesh = plsc.VectorSubcoreMesh(core_axis_name="c", subcore_axis_name="s",
                                  num_cores=_NC, num_subcores=_NS)

    @functools.partial(
        pl.kernel,
        out_type=out_type,
        mesh=mesh,
        compiler_params=pltpu.CompilerParams(use_tc_tiling_on_sc=False),
        scratch_types=[
            pltpu.VMEM((_CH,), jnp.int32),
            pltpu.VMEM((_CH, e), jnp.float32),
            pltpu.VMEM((_CH,), jnp.float32),
            pltpu.SemaphoreType.DMA,
        ],
    )
    def run(*refs):
        tab_refs = refs[0:3]
        ub_r, ib_r = refs[3], refs[4]
        idx_refs = refs[5:11]
        out_refs = refs[11:17]
        ubg_o, ibg_o = refs[17], refs[18]
        idx_v, rows_v, val_v, sem = refs[19:23]

        wid = lax.axis_index("s") * _NC + lax.axis_index("c")

        def row_gather(tab, idx_hbm, out_hbm, nchunk):
            base0 = wid * (nchunk * _CH)

            def body(i, carry):
                off = base0 + i * _CH
                pltpu.sync_copy(idx_hbm.at[pl.ds(off, _CH)], idx_v)
                pltpu.async_copy(tab.at[idx_v], rows_v, sem).wait()
                pltpu.sync_copy(rows_v, out_hbm.at[pl.ds(off, _CH)])
                return carry

            lax.fori_loop(0, nchunk, body, 0)

        for (ti, _), ix_ref, o_ref, nc in zip(row_jobs, idx_refs, out_refs,
                                              nchunks):
            row_gather(tab_refs[ti], ix_ref, o_ref, nc)

        def elem_gather(tab, idx_hbm, out_hbm):
            off = wid * _CH
            pltpu.sync_copy(idx_hbm.at[pl.ds(off, _CH)], idx_v)
            pltpu.async_copy(tab.at[idx_v], val_v, sem).wait()
            pltpu.sync_copy(val_v, out_hbm.at[pl.ds(off, _CH)])

        elem_gather(ub_r, idx_refs[5], ubg_o)
        elem_gather(ib_r, idx_refs[4], ibg_o)

    return run(dwsc_tab, emb_tab, user_tab, ubias, ibias,
               hop_idx, self_idx, ih_idx, uh_idx, iidx, uidx)


def _tc_forward(u_rows, ih3, uh3, ie, dh4, ds3, ub2, ib2,
                isc6, ioc6, usc6, uoc6, genre, gender, age, occ, miu2,
                wb, B, NB, E):
    BB = 128
    H = BB // 2
    G = B // BB
    PH = H * NB

    def body(u_ref, ih_ref, uh_ref, ie_ref, dh_ref, ds_ref, ub_ref, ib_ref,
             isc_ref, ioc_ref, usc_ref, uoc_ref,
             g_ref, gen_ref, age_ref, occ_ref, miu_ref, *rest):
        out_ref = rest[-1]
        wrefs = rest[:-1]
        wd = {}
        for k, name in enumerate(_LAYERS):
            wd[name] = (wrefs[2 * k][...], wrefs[2 * k + 1][...])

        def dot(x, w):
            return jax.lax.dot_general(
                x, w, (((1,), (0,)), ((), ())),
                preferred_element_type=jnp.float32)

        def bd(w):
            k, n = w.shape
            z = jnp.zeros((k, n), jnp.float32)
            return jnp.concatenate(
                [jnp.concatenate([w, z], axis=1),
                 jnp.concatenate([z, w], axis=1)], axis=0)

        def lin(x, name):
            w, bb = wd[name]
            return dot(x, bd(w)) + jnp.concatenate([bb, bb], axis=1)

        def lin2(x1, x2, name):
            w, bb = wd[name]
            return (dot(x1, bd(w[:E, :])) + dot(x2, bd(w[E:, :]))
                    + jnp.concatenate([bb, bb], axis=1))

        def leaky(x):
            return jnp.where(x >= 0, x, 0.01 * x)

        def sig(x):
            return 1.0 / (1.0 + jnp.exp(-x))

        def interact(S, Q, bi, si):
            deep = 0.5 * (S * S - Q)
            return leaky(lin(deep, bi)) + leaky(lin(S, si))

        def sumsq0(v):
            return v.sum(axis=0), (v * v).sum(axis=0)

        def counts_packed(idx2k, V):
            n, k2 = idx2k.shape
            k = k2 // 2
            col = lax.broadcasted_iota(jnp.int32, (n, 2 * V), 1)
            colv = jnp.where(col < V, col, col - V)
            even = col < V
            c = jnp.zeros((n, 2 * V), jnp.float32)
            one = jnp.float32(1.0)
            zero = jnp.float32(0.0)
            for s in range(k):
                c = c + jnp.where(even & (idx2k[:, s:s + 1] == colv), one, zero)
                c = c + jnp.where((~even) & (idx2k[:, k + s:k + s + 1] == colv),
                                  one, zero)
            return c

        g = g_ref[...]
        g2 = g * g
        NG = g.shape[0]

        Sd, Qd = sumsq0(ds_ref[...])
        cg = counts_packed(isc_ref[...], NG)
        att1 = interact(Sd + dot(cg, bd(g)), Qd + dot(cg, bd(g2)),
                        'dense_item_self_biinter', 'dense_item_self_siinter')
        item_cat = jnp.maximum(lin2(ie_ref[...], att1, 'dense_item_cate_self'), 0.0)

        Sh, Qh = sumsq0(dh_ref[...].reshape(8, PH, 2 * E))
        chg = counts_packed(ioc_ref[...].reshape(PH, 6), NG)
        att2 = interact(Sh + dot(chg, bd(g)), Qh + dot(chg, bd(g2)),
                        'dense_item_onehop_biinter', 'dense_item_onehop_siinter')
        inb = ih_ref[...].reshape(PH, 2 * E)
        item_nb = jnp.maximum(lin2(inb, att2, 'dense_item_cate_hop1'), 0.0)
        item_nb_agg = item_nb.reshape(NB, H, 2 * E).sum(axis=0) * (1.0 / NB)

        add_g = sig(lin2(item_cat, item_nb_agg, 'dense_item_addgate'))
        erase_g = sig(lin2(item_cat, item_nb_agg, 'dense_item_erasegate'))
        item_final = item_cat * (1.0 - erase_g) + item_nb_agg * add_g

        usc_v = usc_ref[...]

        def one_tab(s, tab):
            V = tab.shape[0]
            idx2 = jnp.concatenate(
                [usc_v[:, s:s + 1], usc_v[:, 3 + s:4 + s]], axis=1)
            cp = counts_packed(idx2, V)
            return dot(cp, bd(tab)), dot(cp, bd(tab * tab))

        Sg, Qg = one_tab(0, gen_ref[...])
        Sa, Qa = one_tab(1, age_ref[...])
        So, Qo = one_tab(2, occ_ref[...])
        att3 = interact(Sg + Sa + So, Qg + Qa + Qo,
                        'dense_user_self_biinter', 'dense_user_self_siinter')
        user_cat = jnp.maximum(lin2(u_ref[...], att3, 'dense_user_cate_self'), 0.0)

        cu = counts_packed(uoc_ref[...].reshape(PH, 6), NG)
        att4 = interact(dot(cu, bd(g)), dot(cu, bd(g2)),
                        'dense_user_onehop_biinter', 'dense_user_onehop_siinter')
        unb = uh_ref[...].reshape(PH, 2 * E)
        user_nb = jnp.maximum(lin2(unb, att4, 'dense_user_cate_hop1'), 0.0)
        user_nb_agg = user_nb.reshape(NB, H, 2 * E).sum(axis=0) * (1.0 / NB)

        uadd = sig(lin2(user_cat, user_nb_agg, 'dense_user_addgate'))
        uerase = sig(lin2(user_cat, user_nb_agg, 'dense_user_erasegate'))
        user_final = user_cat * (1.0 - uerase) + user_nb_agg * uadd

        wfc, bfc = wd['FC_pre']
        wu = wfc[:, :E]
        wi = wfc[:, E:]
        const = bfc[0, 0] + miu_ref[0, 0]
        pe = ((user_final[:, :E] * wu).sum(axis=1)
              + (item_final[:, :E] * wi).sum(axis=1)
              + ub_ref[:, 0] + ib_ref[:, 0] + const)
        po = ((user_final[:, E:] * wu).sum(axis=1)
              + (item_final[:, E:] * wi).sum(axis=1)
              + ub_ref[:, 1] + ib_ref[:, 1] + const)
        out_ref[...] = jnp.stack([pe, po], axis=1)

    def full(a):
        return pl.BlockSpec(a.shape, lambda i: (0, 0))

    in_specs = [
        pl.BlockSpec((H, 2 * E), lambda i: (i, 0)),
        pl.BlockSpec((NB, H, 2 * E), lambda i: (0, i, 0)),
        pl.BlockSpec((NB, H, 2 * E), lambda i: (0, i, 0)),
        pl.BlockSpec((H, 2 * E), lambda i: (i, 0)),
        pl.BlockSpec((8, NB, H, 2 * E), lambda i: (0, 0, i, 0)),
        pl.BlockSpec((8, H, 2 * E), lambda i: (0, i, 0)),
        pl.BlockSpec((H, 2), lambda i: (i, 0)),
        pl.BlockSpec((H, 2), lambda i: (i, 0)),
        pl.BlockSpec((H, 6), lambda i: (i, 0)),
        pl.BlockSpec((NB, H, 6), lambda i: (0, i, 0)),
        pl.BlockSpec((H, 6), lambda i: (i, 0)),
        pl.BlockSpec((NB, H, 6), lambda i: (0, i, 0)),
        full(genre), full(gender), full(age), full(occ), full(miu2),
    ]
    ops = [u_rows, ih3, uh3, ie, dh4, ds3, ub2, ib2,
           isc6, ioc6, usc6, uoc6, genre, gender, age, occ, miu2]
    for w, bb_ in wb:
        in_specs.append(full(w))
        in_specs.append(full(bb_))
        ops.append(w)
        ops.append(bb_)

    out = pl.pallas_call(
        body,
        grid=(G,),
        in_specs=in_specs,
        out_specs=pl.BlockSpec((H, 2), lambda i: (i, 0)),
        out_shape=jax.ShapeDtypeStruct((B // 2, 2), jnp.float32),
    )(*ops)
    return out.reshape(B)


def kernel(params, user, item, user_self_cate, user_onehop_id, user_onehop_cate,
           item_self_cate, item_self_director, item_self_writer, item_self_star,
           item_self_country, item_onehop_id, item_onehop_cate,
           item_onehop_director, item_onehop_writer, item_onehop_star,
           item_onehop_country):
    p = params
    B = user.shape[0]
    NB = user_onehop_id.shape[1]
    E = p['user_embed'].shape[1]

    d_tab = p['director_embed']
    w_tab = p['writer_embed']
    s_tab = p['star_embed']
    c_tab = p['country_embed']
    off_w = d_tab.shape[0]
    off_s = off_w + w_tab.shape[0]
    off_c = off_s + s_tab.shape[0]
    dwsc_tab = jnp.concatenate([d_tab, w_tab, s_tab, c_tab], axis=0)

    def tslot(a, k, off):
        return (a[:, :, k] + off).transpose(1, 0).reshape(-1)

    hop_idx = jnp.concatenate([
        tslot(item_onehop_director, 0, 0), tslot(item_onehop_director, 1, 0),
        tslot(item_onehop_writer, 0, off_w), tslot(item_onehop_writer, 1, off_w),
        tslot(item_onehop_star, 0, off_s), tslot(item_onehop_star, 1, off_s),
        tslot(item_onehop_star, 2, off_s),
        tslot(item_onehop_country, 0, off_c)])
    self_idx = jnp.concatenate([
        item_self_director[:, 0], item_self_director[:, 1],
        item_self_writer[:, 0] + off_w, item_self_writer[:, 1] + off_w,
        item_self_star[:, 0] + off_s, item_self_star[:, 1] + off_s,
        item_self_star[:, 2] + off_s,
        item_self_country[:, 0] + off_c])
    ih_idx = item_onehop_id.transpose(1, 0).reshape(-1)
    uh_idx = user_onehop_id.transpose(1, 0).reshape(-1)

    (hop_rows, self_rows, ih_rows, uh_rows, ie, u_rows,
     ub_g, ib_g) = _sc_gather(
        dwsc_tab, p['item_embed'], p['user_embed'],
        p['user_bias'].reshape(-1), p['item_bias'].reshape(-1),
        hop_idx, self_idx, ih_idx, uh_idx, item, user)

    dh4 = hop_rows.reshape(8, NB, B // 2, 2 * E)
    ds3 = self_rows.reshape(8, B // 2, 2 * E)
    ih3 = ih_rows.reshape(NB, B // 2, 2 * E)
    uh3 = uh_rows.reshape(NB, B // 2, 2 * E)
    iep = ie.reshape(B // 2, 2 * E)
    up = u_rows.reshape(B // 2, 2 * E)

    wb = []
    for name in _LAYERS[:-1]:
        wb.append((p[name + '_w'], p[name + '_b'].reshape(1, -1)))
    wb.append((p['FC_pre_w'].reshape(1, 2 * E), p['FC_pre_b'].reshape(1, 1)))

    return _tc_forward(
        up, ih3, uh3, iep, dh4, ds3,
        ub_g.reshape(B // 2, 2), ib_g.reshape(B // 2, 2),
        item_self_cate.reshape(B // 2, 6),
        item_onehop_cate.transpose(1, 0, 2).reshape(NB, B // 2, 6),
        user_self_cate.reshape(B // 2, 6),
        user_onehop_cate.transpose(1, 0, 2).reshape(NB, B // 2, 6),
        p['genre_embed'], p['gender_embed'], p['age_embed'],
        p['occupation_embed'], p['miu'].reshape(1, 1),
        wb, B, NB, E)


# speedup vs baseline: 4.5779x; 1.1033x over previous
import functools

import jax
import jax.numpy as jnp
from jax import lax
from jax.experimental import pallas as pl
from jax.experimental.pallas import tpu as pltpu
from jax.experimental.pallas import tpu_sc as plsc

_NC, _NS = 2, 16
_NW = _NC * _NS
_CH = 128

_LAYERS = (
    'dense_item_self_biinter', 'dense_item_self_siinter',
    'dense_item_onehop_biinter', 'dense_item_onehop_siinter',
    'dense_user_self_biinter', 'dense_user_self_siinter',
    'dense_user_onehop_biinter', 'dense_user_onehop_siinter',
    'dense_item_cate_self', 'dense_item_cate_hop1',
    'dense_user_cate_self', 'dense_user_cate_hop1',
    'dense_item_addgate', 'dense_item_erasegate',
    'dense_user_addgate', 'dense_user_erasegate',
    'FC_pre',
)


def _sc_gather(dwsc_tab, emb_tab, user_tab, ubias, ibias,
               hop_idx, self_idx, ih_idx, uh_idx, iidx, uidx):
    e = emb_tab.shape[1]
    b = uidx.shape[0]

    row_jobs = [
        (0, hop_idx), (0, self_idx),
        (1, ih_idx), (1, uh_idx), (1, iidx), (2, uidx),
    ]
    nchunks = [ix.shape[0] // (_NW * _CH) for _, ix in row_jobs]

    out_type = [jax.ShapeDtypeStruct((ix.shape[0], e), jnp.float32)
                for _, ix in row_jobs]
    out_type += [jax.ShapeDtypeStruct((b,), jnp.float32),
                 jax.ShapeDtypeStruct((b,), jnp.float32)]

    mesh = plsc.VectorSubcoreMesh(core_axis_name="c", subcore_axis_name="s",
                                  num_cores=_NC, num_subcores=_NS)

    @functools.partial(
        pl.kernel,
        out_type=out_type,
        mesh=mesh,
        compiler_params=pltpu.CompilerParams(use_tc_tiling_on_sc=False),
        scratch_types=[
            pltpu.VMEM((4 * _CH,), jnp.int32),
            pltpu.VMEM((4, _CH, e), jnp.float32),
            pltpu.VMEM((_CH,), jnp.float32),
            pltpu.SemaphoreType.DMA,
            pltpu.SemaphoreType.DMA,
        ],
    )
    def run(*refs):
        tab_refs = refs[0:3]
        ub_r, ib_r = refs[3], refs[4]
        idx_refs = refs[5:11]
        out_refs = refs[11:17]
        ubg_o, ibg_o = refs[17], refs[18]
        idx4, rows4, val_v, sem_g, sem_s = refs[19:24]

        wid = lax.axis_index("s") * _NC + lax.axis_index("c")

        def row_gather(tab, idx_hbm, out_hbm, nchunk):
            base0 = wid * (nchunk * _CH)

            if nchunk == 1:
                pltpu.sync_copy(idx_hbm.at[pl.ds(base0, _CH)],
                                idx4.at[pl.ds(0, _CH)])
                pltpu.async_copy(tab.at[idx4.at[pl.ds(0, _CH)]],
                                 rows4.at[0], sem_g).wait()
                pltpu.sync_copy(rows4.at[0], out_hbm.at[pl.ds(base0, _CH)])
                return

            def body(q, carry):
                off = base0 + q * (4 * _CH)
                pltpu.sync_copy(idx_hbm.at[pl.ds(off, 4 * _CH)], idx4)
                gd = [pltpu.async_copy(
                    tab.at[idx4.at[pl.ds(k * _CH, _CH)]], rows4.at[k], sem_g)
                    for k in range(4)]
                for d in gd:
                    d.wait()
                sd = [pltpu.async_copy(
                    rows4.at[k], out_hbm.at[pl.ds(off + k * _CH, _CH)], sem_s)
                    for k in range(4)]
                for d in sd:
                    d.wait()
                return carry

            lax.fori_loop(0, nchunk // 4, body, 0)

        for (ti, _), ix_ref, o_ref, nc in zip(row_jobs, idx_refs, out_refs,
                                              nchunks):
            row_gather(tab_refs[ti], ix_ref, o_ref, nc)

        def elem_gather(tab, idx_hbm, out_hbm):
            off = wid * _CH
            pltpu.sync_copy(idx_hbm.at[pl.ds(off, _CH)],
                            idx4.at[pl.ds(0, _CH)])
            pltpu.async_copy(tab.at[idx4.at[pl.ds(0, _CH)]], val_v,
                             sem_g).wait()
            pltpu.sync_copy(val_v, out_hbm.at[pl.ds(off, _CH)])

        elem_gather(ub_r, idx_refs[5], ubg_o)
        elem_gather(ib_r, idx_refs[4], ibg_o)

    return run(dwsc_tab, emb_tab, user_tab, ubias, ibias,
               hop_idx, self_idx, ih_idx, uh_idx, iidx, uidx)


def _tc_forward(u_rows, ih3, uh3, ie, dh4, ds3, ub2, ib2,
                isc6, ioc6, usc6, uoc6, genre, gender, age, occ, miu2,
                wb, B, NB, E):
    BB = 128
    H = BB // 2
    G = B // BB
    PH = H * NB

    def body(u_ref, ih_ref, uh_ref, ie_ref, dh_ref, ds_ref, ub_ref, ib_ref,
             isc_ref, ioc_ref, usc_ref, uoc_ref,
             g_ref, gen_ref, age_ref, occ_ref, miu_ref, *rest):
        out_ref = rest[-1]
        wrefs = rest[:-1]
        wd = {}
        for k, name in enumerate(_LAYERS):
            wd[name] = (wrefs[2 * k][...], wrefs[2 * k + 1][...])

        def dot(x, w):
            return jax.lax.dot_general(
                x, w, (((1,), (0,)), ((), ())),
                preferred_element_type=jnp.float32)

        def bd(w):
            k, n = w.shape
            z = jnp.zeros((k, n), jnp.float32)
            return jnp.concatenate(
                [jnp.concatenate([w, z], axis=1),
                 jnp.concatenate([z, w], axis=1)], axis=0)

        def lin(x, name):
            w, bb = wd[name]
            return dot(x, bd(w)) + jnp.concatenate([bb, bb], axis=1)

        def lin2(x1, x2, name):
            w, bb = wd[name]
            return (dot(x1, bd(w[:E, :])) + dot(x2, bd(w[E:, :]))
                    + jnp.concatenate([bb, bb], axis=1))

        def leaky(x):
            return jnp.where(x >= 0, x, 0.01 * x)

        def sig(x):
            return 1.0 / (1.0 + jnp.exp(-x))

        def interact(S, Q, bi, si):
            deep = 0.5 * (S * S - Q)
            return leaky(lin(deep, bi)) + leaky(lin(S, si))

        def sumsq0(v):
            return v.sum(axis=0), (v * v).sum(axis=0)

        def counts_packed(idx2k, V):
            n, k2 = idx2k.shape
            k = k2 // 2
            col = lax.broadcasted_iota(jnp.int32, (n, 2 * V), 1)
            colv = jnp.where(col < V, col, col - V)
            even = col < V
            c = jnp.zeros((n, 2 * V), jnp.float32)
            one = jnp.float32(1.0)
            zero = jnp.float32(0.0)
            for s in range(k):
                c = c + jnp.where(even & (idx2k[:, s:s + 1] == colv), one, zero)
                c = c + jnp.where((~even) & (idx2k[:, k + s:k + s + 1] == colv),
                                  one, zero)
            return c

        g = g_ref[...]
        g2 = g * g
        NG = g.shape[0]

        Sd, Qd = sumsq0(ds_ref[...])
        cg = counts_packed(isc_ref[...], NG)
        att1 = interact(Sd + dot(cg, bd(g)), Qd + dot(cg, bd(g2)),
                        'dense_item_self_biinter', 'dense_item_self_siinter')
        item_cat = jnp.maximum(lin2(ie_ref[...], att1, 'dense_item_cate_self'), 0.0)

        Sh, Qh = sumsq0(dh_ref[...].reshape(8, PH, 2 * E))
        chg = counts_packed(ioc_ref[...].reshape(PH, 6), NG)
        att2 = interact(Sh + dot(chg, bd(g)), Qh + dot(chg, bd(g2)),
                        'dense_item_onehop_biinter', 'dense_item_onehop_siinter')
        inb = ih_ref[...].reshape(PH, 2 * E)
        item_nb = jnp.maximum(lin2(inb, att2, 'dense_item_cate_hop1'), 0.0)
        item_nb_agg = item_nb.reshape(NB, H, 2 * E).sum(axis=0) * (1.0 / NB)

        add_g = sig(lin2(item_cat, item_nb_agg, 'dense_item_addgate'))
        erase_g = sig(lin2(item_cat, item_nb_agg, 'dense_item_erasegate'))
        item_final = item_cat * (1.0 - erase_g) + item_nb_agg * add_g

        usc_v = usc_ref[...]

        def one_tab(s, tab):
            V = tab.shape[0]
            idx2 = jnp.concatenate(
                [usc_v[:, s:s + 1], usc_v[:, 3 + s:4 + s]], axis=1)
            cp = counts_packed(idx2, V)
            return dot(cp, bd(tab)), dot(cp, bd(tab * tab))

        Sg, Qg = one_tab(0, gen_ref[...])
        Sa, Qa = one_tab(1, age_ref[...])
        So, Qo = one_tab(2, occ_ref[...])
        att3 = interact(Sg + Sa + So, Qg + Qa + Qo,
                        'dense_user_self_biinter', 'dense_user_self_siinter')
        user_cat = jnp.maximum(lin2(u_ref[...], att3, 'dense_user_cate_self'), 0.0)

        cu = counts_packed(uoc_ref[...].reshape(PH, 6), NG)
        att4 = interact(dot(cu, bd(g)), dot(cu, bd(g2)),
                        'dense_user_onehop_biinter', 'dense_user_onehop_siinter')
        unb = uh_ref[...].reshape(PH, 2 * E)
        user_nb = jnp.maximum(lin2(unb, att4, 'dense_user_cate_hop1'), 0.0)
        user_nb_agg = user_nb.reshape(NB, H, 2 * E).sum(axis=0) * (1.0 / NB)

        uadd = sig(lin2(user_cat, user_nb_agg, 'dense_user_addgate'))
        uerase = sig(lin2(user_cat, user_nb_agg, 'dense_user_erasegate'))
        user_final = user_cat * (1.0 - uerase) + user_nb_agg * uadd

        wfc, bfc = wd['FC_pre']
        wu = wfc[:, :E]
        wi = wfc[:, E:]
        const = bfc[0, 0] + miu_ref[0, 0]
        pe = ((user_final[:, :E] * wu).sum(axis=1)
              + (item_final[:, :E] * wi).sum(axis=1)
              + ub_ref[:, 0] + ib_ref[:, 0] + const)
        po = ((user_final[:, E:] * wu).sum(axis=1)
              + (item_final[:, E:] * wi).sum(axis=1)
              + ub_ref[:, 1] + ib_ref[:, 1] + const)
        out_ref[...] = jnp.stack([pe, po], axis=1)

    def full(a):
        return pl.BlockSpec(a.shape, lambda i: (0, 0))

    in_specs = [
        pl.BlockSpec((H, 2 * E), lambda i: (i, 0)),
        pl.BlockSpec((NB, H, 2 * E), lambda i: (0, i, 0)),
        pl.BlockSpec((NB, H, 2 * E), lambda i: (0, i, 0)),
        pl.BlockSpec((H, 2 * E), lambda i: (i, 0)),
        pl.BlockSpec((8, NB, H, 2 * E), lambda i: (0, 0, i, 0)),
        pl.BlockSpec((8, H, 2 * E), lambda i: (0, i, 0)),
        pl.BlockSpec((H, 2), lambda i: (i, 0)),
        pl.BlockSpec((H, 2), lambda i: (i, 0)),
        pl.BlockSpec((H, 6), lambda i: (i, 0)),
        pl.BlockSpec((NB, H, 6), lambda i: (0, i, 0)),
        pl.BlockSpec((H, 6), lambda i: (i, 0)),
        pl.BlockSpec((NB, H, 6), lambda i: (0, i, 0)),
        full(genre), full(gender), full(age), full(occ), full(miu2),
    ]
    ops = [u_rows, ih3, uh3, ie, dh4, ds3, ub2, ib2,
           isc6, ioc6, usc6, uoc6, genre, gender, age, occ, miu2]
    for w, bb_ in wb:
        in_specs.append(full(w))
        in_specs.append(full(bb_))
        ops.append(w)
        ops.append(bb_)

    out = pl.pallas_call(
        body,
        grid=(G,),
        in_specs=in_specs,
        out_specs=pl.BlockSpec((H, 2), lambda i: (i, 0)),
        out_shape=jax.ShapeDtypeStruct((B // 2, 2), jnp.float32),
    )(*ops)
    return out.reshape(B)


def kernel(params, user, item, user_self_cate, user_onehop_id, user_onehop_cate,
           item_self_cate, item_self_director, item_self_writer, item_self_star,
           item_self_country, item_onehop_id, item_onehop_cate,
           item_onehop_director, item_onehop_writer, item_onehop_star,
           item_onehop_country):
    p = params
    B = user.shape[0]
    NB = user_onehop_id.shape[1]
    E = p['user_embed'].shape[1]

    d_tab = p['director_embed']
    w_tab = p['writer_embed']
    s_tab = p['star_embed']
    c_tab = p['country_embed']
    off_w = d_tab.shape[0]
    off_s = off_w + w_tab.shape[0]
    off_c = off_s + s_tab.shape[0]
    dwsc_tab = jnp.concatenate([d_tab, w_tab, s_tab, c_tab], axis=0)

    def tslot(a, k, off):
        return (a[:, :, k] + off).transpose(1, 0).reshape(-1)

    hop_idx = jnp.concatenate([
        tslot(item_onehop_director, 0, 0), tslot(item_onehop_director, 1, 0),
        tslot(item_onehop_writer, 0, off_w), tslot(item_onehop_writer, 1, off_w),
        tslot(item_onehop_star, 0, off_s), tslot(item_onehop_star, 1, off_s),
        tslot(item_onehop_star, 2, off_s),
        tslot(item_onehop_country, 0, off_c)])
    self_idx = jnp.concatenate([
        item_self_director[:, 0], item_self_director[:, 1],
        item_self_writer[:, 0] + off_w, item_self_writer[:, 1] + off_w,
        item_self_star[:, 0] + off_s, item_self_star[:, 1] + off_s,
        item_self_star[:, 2] + off_s,
        item_self_country[:, 0] + off_c])
    ih_idx = item_onehop_id.transpose(1, 0).reshape(-1)
    uh_idx = user_onehop_id.transpose(1, 0).reshape(-1)

    (hop_rows, self_rows, ih_rows, uh_rows, ie, u_rows,
     ub_g, ib_g) = _sc_gather(
        dwsc_tab, p['item_embed'], p['user_embed'],
        p['user_bias'].reshape(-1), p['item_bias'].reshape(-1),
        hop_idx, self_idx, ih_idx, uh_idx, item, user)

    dh4 = hop_rows.reshape(8, NB, B // 2, 2 * E)
    ds3 = self_rows.reshape(8, B // 2, 2 * E)
    ih3 = ih_rows.reshape(NB, B // 2, 2 * E)
    uh3 = uh_rows.reshape(NB, B // 2, 2 * E)
    iep = ie.reshape(B // 2, 2 * E)
    up = u_rows.reshape(B // 2, 2 * E)

    wb = []
    for name in _LAYERS[:-1]:
        wb.append((p[name + '_w'], p[name + '_b'].reshape(1, -1)))
    wb.append((p['FC_pre_w'].reshape(1, 2 * E), p['FC_pre_b'].reshape(1, 1)))

    return _tc_forward(
        up, ih3, uh3, iep, dh4, ds3,
        ub_g.reshape(B // 2, 2), ib_g.reshape(B // 2, 2),
        item_self_cate.reshape(B // 2, 6),
        item_onehop_cate.transpose(1, 0, 2).reshape(NB, B // 2, 6),
        user_self_cate.reshape(B // 2, 6),
        user_onehop_cate.transpose(1, 0, 2).reshape(NB, B // 2, 6),
        p['genre_embed'], p['gender_embed'], p['age_embed'],
        p['occupation_embed'], p['miu'].reshape(1, 1),
        wb, B, NB, E)
